# baseline (speedup 1.0000x reference)
import functools
import math

import jax
import jax.numpy as jnp
import numpy as np
from jax import lax
from jax.experimental import pallas as pl
from jax.experimental.pallas import tpu as pltpu
from jax.experimental.pallas import tpu_sc as plsc

D_MODEL = 1024
LANES = 16
NSLICE = D_MODEL // LANES
NC = 2
NS = 16
NW = NC * NS
CHUNK = 32


def _pe_table(seq_len: int, d_model: int) -> np.ndarray:
    position = np.arange(seq_len, dtype=np.float32)[:, None]
    div_term = np.exp(
        np.arange(0, d_model, 2, dtype=np.float32) * (-math.log(10000.0) / d_model)
    )
    angles = position * div_term[None, :]
    pe = np.zeros((seq_len, d_model), dtype=np.float32)
    pe[:, 0::2] = np.sin(angles)
    pe[:, 1::2] = np.cos(angles)
    return pe


def _sc_embed_ln(idx, W, pe, gamma, beta):
    B = idx.shape[0]
    S = pe.shape[0]
    BPW = B // NW
    NCH = BPW // CHUNK
    scale = float(math.sqrt(D_MODEL))

    mesh = plsc.VectorSubcoreMesh(core_axis_name="c", subcore_axis_name="s")

    gdn = lax.GatherDimensionNumbers(
        offset_dims=(), collapsed_slice_dims=(0,), start_index_map=(0,))

    def _lane_perm(v, p):
        return lax.gather(
            v, p[:, None], dimension_numbers=gdn, slice_sizes=(1,),
            mode=lax.GatherScatterMode.PROMISE_IN_BOUNDS)

    def _allsum(v):
        lane = lax.iota(jnp.int32, LANES)
        for k in range(4):
            v = v + _lane_perm(v, lane ^ (1 << k))
        return v

    @functools.partial(
        pl.kernel,
        mesh=mesh,
        out_type=jax.ShapeDtypeStruct((B, D_MODEL), jnp.float32),
        scratch_types=[
            pltpu.VMEM((BPW,), jnp.int32),
            pltpu.VMEM((CHUNK, D_MODEL), jnp.float32),
            pltpu.VMEM((CHUNK, D_MODEL), jnp.float32),
            pltpu.VMEM((D_MODEL,), jnp.float32),
            pltpu.VMEM((D_MODEL,), jnp.float32),
            pltpu.SemaphoreType.DMA,
        ],
    )
    def body(idx_hbm, w_hbm, pe_hbm, g_hbm, b_hbm, out_hbm,
             idx_v, rows_v, pe_v, g_v, b_v, sem):
        cid = lax.axis_index("c")
        sid = lax.axis_index("s")
        wid = sid * NC + cid
        base = wid * BPW
        pltpu.sync_copy(idx_hbm.at[pl.ds(base, BPW)], idx_v)
        pltpu.sync_copy(g_hbm, g_v)
        pltpu.sync_copy(b_hbm, b_v)
        for ch in range(NCH):
            rbase = base + ch * CHUNK
            pb = lax.rem(rbase, S)
            pltpu.async_copy(
                w_hbm.at[idx_v.at[pl.ds(ch * CHUNK, CHUNK)]], rows_v, sem
            ).wait()
            pltpu.sync_copy(pe_hbm.at[pl.ds(pb, CHUNK)], pe_v)

            def row_body(r, carry):
                sv = jnp.zeros((LANES,), jnp.float32)
                qv = jnp.zeros((LANES,), jnp.float32)
                for j in range(NSLICE):
                    sl = pl.ds(j * LANES, LANES)
                    h = rows_v[r, sl] * scale + pe_v[r, sl]
                    rows_v[r, sl] = h
                    sv = sv + h
                    qv = qv + h * h
                mu_v = _allsum(sv) * (1.0 / D_MODEL)
                var_v = _allsum(qv) * (1.0 / D_MODEL) - mu_v * mu_v
                xv = var_v + 1e-5
                bits = lax.bitcast_convert_type(xv, jnp.int32)
                y = lax.bitcast_convert_type(
                    jnp.full((LANES,), 0x5F3759DF, jnp.int32) - (bits >> 1),
                    jnp.float32)
                for _ in range(3):
                    y = y * (1.5 - 0.5 * xv * y * y)
                for j in range(NSLICE):
                    sl = pl.ds(j * LANES, LANES)
                    h = rows_v[r, sl]
                    rows_v[r, sl] = (h - mu_v) * y * g_v[sl] + b_v[sl]
                return carry

            lax.fori_loop(0, CHUNK, row_body, 0)
            pltpu.sync_copy(rows_v, out_hbm.at[pl.ds(rbase, CHUNK)])

    return body(idx, W, pe, gamma, beta)


def kernel(x, W, gamma, beta):
    bsz, seq = x.shape
    idx = x.reshape(-1).astype(jnp.int32)
    pe = jnp.asarray(_pe_table(seq, D_MODEL))
    out = _sc_embed_ln(idx, W, pe, gamma, beta)
    return out.reshape(bsz, seq, D_MODEL)
